# baseline (speedup 1.0000x reference)
import functools

import jax
import jax.numpy as jnp
from jax import lax
from jax.experimental import pallas as pl
from jax.experimental.pallas import tpu as pltpu
from jax.experimental.pallas import tpu_sc as plsc

BATCH = 16384
EMB = 64
NUM_NUMERIC = 12
NUM_STYLES = 50

_NC = 2
_NS = 16
_NW = _NC * _NS
_BPW = BATCH // _NW
_CHUNK = 128
_NCHUNK = _BPW // _CHUNK


def _sc_gather(user_id, product_id, user_table, product_table):
    mesh = plsc.VectorSubcoreMesh(core_axis_name="c", subcore_axis_name="s")

    @functools.partial(
        pl.kernel,
        mesh=mesh,
        out_type=(
            jax.ShapeDtypeStruct((BATCH, EMB), jnp.float32),
            jax.ShapeDtypeStruct((BATCH, EMB), jnp.float32),
        ),
        scratch_types=[
            pltpu.VMEM((_NCHUNK, _CHUNK), jnp.int32),
            pltpu.VMEM((_NCHUNK, _CHUNK), jnp.int32),
            pltpu.VMEM((_BPW, EMB), jnp.float32),
            pltpu.VMEM((_BPW, EMB), jnp.float32),
            pltpu.SemaphoreType.DMA,
        ],
        compiler_params=pltpu.CompilerParams(use_tc_tiling_on_sc=False),
    )
    def k(uid_hbm, pid_hbm, utab_hbm, ptab_hbm, uout_hbm, pout_hbm,
          uidx_v, pidx_v, urows_v, prows_v, sem):
        wid = lax.axis_index("s") * _NC + lax.axis_index("c")
        base = wid * _BPW
        pltpu.sync_copy(uid_hbm.at[wid], uidx_v)
        pltpu.sync_copy(pid_hbm.at[wid], pidx_v)
        copies = []
        for j in range(_NCHUNK):
            copies.append(pltpu.async_copy(
                utab_hbm.at[uidx_v.at[j]],
                urows_v.at[pl.ds(j * _CHUNK, _CHUNK)], sem))
            copies.append(pltpu.async_copy(
                ptab_hbm.at[pidx_v.at[j]],
                prows_v.at[pl.ds(j * _CHUNK, _CHUNK)], sem))
        for c in copies:
            c.wait()
        pltpu.sync_copy(urows_v, uout_hbm.at[pl.ds(base, _BPW)])
        pltpu.sync_copy(prows_v, pout_hbm.at[pl.ds(base, _BPW)])

    uid3 = user_id.reshape(_NW, _NCHUNK, _CHUNK)
    pid3 = product_id.reshape(_NW, _NCHUNK, _CHUNK)
    return k(uid3, pid3, user_table, product_table)


def _mlp_body(u_ref, p_ref, ff_ref, Wn_ref, bn_ref, Ws_ref, bs_ref,
              W1u_ref, W1p_ref, W1n_ref, W1s_ref, b1_ref,
              W2_ref, b2_ref, W3_ref, b3_ref, wf_ref, bf_ref, o_ref):
    f32 = jnp.float32
    ff = ff_ref[...]
    nvec = jnp.maximum(jnp.dot(ff, Wn_ref[...], preferred_element_type=f32)
                       + bn_ref[...], 0.0)
    svec = jnp.maximum(jnp.dot(ff, Ws_ref[...], preferred_element_type=f32)
                       + bs_ref[...], 0.0)
    h = (jnp.dot(u_ref[...], W1u_ref[...], preferred_element_type=f32)
         + jnp.dot(p_ref[...], W1p_ref[...], preferred_element_type=f32)
         + jnp.dot(nvec, W1n_ref[...], preferred_element_type=f32)
         + jnp.dot(svec, W1s_ref[...], preferred_element_type=f32)
         + b1_ref[...])
    h = jnp.maximum(h, 0.0)
    x2 = jnp.maximum(jnp.dot(h, W2_ref[...], preferred_element_type=f32)
                     + b2_ref[...], 0.0)
    x3 = jnp.maximum(jnp.dot(x2, W3_ref[...], preferred_element_type=f32)
                     + b3_ref[...], 0.0)
    logit = jnp.sum(x3 * wf_ref[...], axis=1, keepdims=True) + bf_ref[...]
    o_ref[...] = jax.nn.sigmoid(logit)


def _mlp(uvec, pvec, ffp, Wn, bn, Ws, bs, W1u, W1p, W1n, W1s, b1,
         W2, b2, W3, b3, wf_row, bf):
    R = 2048
    grid = (BATCH // R,)

    def rows(i):
        return (i, 0)

    def whole(i):
        return (0, 0)

    row_spec = lambda w: pl.BlockSpec((R, w), rows)
    full_spec = lambda a: pl.BlockSpec(a.shape, whole)

    return pl.pallas_call(
        _mlp_body,
        grid=grid,
        in_specs=[
            row_spec(EMB), row_spec(EMB), row_spec(64),
            full_spec(Wn), full_spec(bn), full_spec(Ws), full_spec(bs),
            full_spec(W1u), full_spec(W1p), full_spec(W1n), full_spec(W1s),
            full_spec(b1), full_spec(W2), full_spec(b2),
            full_spec(W3), full_spec(b3), full_spec(wf_row), full_spec(bf),
        ],
        out_specs=pl.BlockSpec((R, 1), rows),
        out_shape=jax.ShapeDtypeStruct((BATCH, 1), jnp.float32),
    )(uvec, pvec, ffp, Wn, bn, Ws, bs, W1u, W1p, W1n, W1s, b1,
      W2, b2, W3, b3, wf_row, bf)


def kernel(user_id, product_id, full_features, user_table, product_table,
           W_num, b_num, W_style, b_style, W1, b1, W2, b2, W3, b3, Wf, bf):
    uid = user_id.astype(jnp.int32)
    pid = product_id.astype(jnp.int32)

    uvec, pvec = _sc_gather(uid, pid, user_table, product_table)

    ffp = jnp.pad(full_features, ((0, 0), (0, 2)))
    Wn = jnp.zeros((64, EMB), jnp.float32).at[:NUM_NUMERIC].set(W_num)
    Ws = jnp.zeros((64, EMB), jnp.float32).at[
        NUM_NUMERIC:NUM_NUMERIC + NUM_STYLES].set(W_style)

    W1u = W1[:EMB]
    W1p = W1[EMB:2 * EMB]
    W1n = W1[2 * EMB:3 * EMB]
    W1s = W1[3 * EMB:]

    return _mlp(uvec, pvec, ffp,
                Wn, b_num.reshape(1, EMB), Ws, b_style.reshape(1, EMB),
                W1u, W1p, W1n, W1s, b1.reshape(1, 128),
                W2, b2.reshape(1, 64), W3, b3.reshape(1, 32),
                Wf.reshape(1, 32), bf.reshape(1, 1))
